# baseline (speedup 1.0000x reference)
import jax
import jax.numpy as jnp
from jax.experimental import pallas as pl

_GRID = 25


def _copy_body(x_ref, e_ref, ox_ref, oe_ref):
    ox_ref[...] = x_ref[...]
    oe_ref[...] = e_ref[...]


def kernel(x, edge_index, edge_attr):
    del edge_index
    n_nodes, d_feat = x.shape
    n_edges, d_edge = edge_attr.shape
    bx = n_nodes // _GRID
    be = n_edges // _GRID
    out = pl.pallas_call(
        _copy_body,
        grid=(_GRID,),
        in_specs=[
            pl.BlockSpec((bx, d_feat), lambda i: (i, 0)),
            pl.BlockSpec((be, d_edge), lambda i: (i, 0)),
        ],
        out_specs=[
            pl.BlockSpec((bx, d_feat), lambda i: (i, 0)),
            pl.BlockSpec((be, d_edge), lambda i: (i, 0)),
        ],
        out_shape=[
            jax.ShapeDtypeStruct(x.shape, x.dtype),
            jax.ShapeDtypeStruct(edge_attr.shape, edge_attr.dtype),
        ],
    )(x, edge_attr)
    return (out[0], out[1])
